# baseline (speedup 1.0000x reference)
import functools

import numpy as np
import jax
import jax.numpy as jnp
from jax import lax
from jax.experimental import pallas as pl
from jax.experimental.pallas import tpu as pltpu
from jax.experimental.pallas import tpu_sc as plsc


@functools.lru_cache(maxsize=None)
def _pos_factor_table(S: int, D: int, NW: int, C: int):
    SW = S // NW
    NCH = SW // C
    i = np.arange(0, D, 2, dtype=np.float64)
    freq = np.repeat(1.0 / np.power(10000.0, i / D), 2)
    off = np.tile(np.array([0.0, np.pi / 2]), D // 2)
    w = np.arange(NW, dtype=np.float64)[:, None]
    g = np.arange(NCH, dtype=np.float64)[:, None]
    r = np.arange(C, dtype=np.float64)[:, None]
    alpha = w * SW * freq[None, :] + off[None, :]
    beta = g * C * freq[None, :]
    gamma = r * freq[None, :]
    tab = np.concatenate([
        np.sin(alpha), np.cos(alpha),
        np.sin(beta), np.cos(beta),
        np.sin(gamma), np.cos(gamma),
    ]).astype(np.float32)
    return jnp.asarray(tab)


@functools.lru_cache(maxsize=None)
def _build_sc_kernel(B: int, S: int, V: int, D: int):
    info = plsc.get_sparse_core_info()
    NC, NS, L = info.num_cores, info.num_subcores, info.num_lanes
    NW = NC * NS
    SW = S // NW
    C = 8
    NCH = SW // C
    NVEC = D // L

    mesh = plsc.VectorSubcoreMesh(core_axis_name="c", subcore_axis_name="s")

    @functools.partial(
        pl.kernel,
        out_type=jax.ShapeDtypeStruct((B * S, D), jnp.float32),
        mesh=mesh,
        scratch_types=[
            pltpu.VMEM((B, SW), jnp.int32),
            pltpu.VMEM((2, B, C, D), jnp.float32),
            pltpu.VMEM((1, D), jnp.float32),
            pltpu.VMEM((1, D), jnp.float32),
            pltpu.VMEM((2 * NCH, D), jnp.float32),
            pltpu.VMEM((2 * C, D), jnp.float32),
            pltpu.VMEM((D,), jnp.float32),
            pltpu.VMEM((D,), jnp.float32),
            pltpu.SemaphoreType.DMA((2,)),
            pltpu.SemaphoreType.DMA((2,)),
        ],
    )
    def k(x_hbm, tab_hbm, table_hbm, out_hbm, idx_v, rows_v,
          bs_v, bc_v, gsc_v, rsc_v, bgs_v, bgc_v, gsem, ssem):
        wid = lax.axis_index("s") * NC + lax.axis_index("c")
        s_base = wid * SW
        for b in range(B):
            pltpu.sync_copy(x_hbm.at[b, pl.ds(s_base, SW)], idx_v.at[b])
        pltpu.sync_copy(tab_hbm.at[pl.ds(wid, 1)], bs_v)
        pltpu.sync_copy(tab_hbm.at[pl.ds(NW + wid, 1)], bc_v)
        pltpu.sync_copy(tab_hbm.at[pl.ds(2 * NW, 2 * NCH)], gsc_v)
        pltpu.sync_copy(tab_hbm.at[pl.ds(2 * NW + 2 * NCH, 2 * C)], rsc_v)

        def in_copies(g, p):
            for b in range(B):
                yield pltpu.make_async_copy(
                    table_hbm.at[idx_v.at[b, pl.ds(g * C, C)]],
                    rows_v.at[p, b], gsem.at[p])

        def out_copies(g, p):
            s0 = s_base + g * C
            for b in range(B):
                yield pltpu.make_async_copy(
                    rows_v.at[p, b], out_hbm.at[pl.ds(b * S + s0, C)],
                    ssem.at[p])

        for c in in_copies(0, 0):
            c.start()

        def step(g, p):
            @pl.when(g >= 1)
            def _():
                for c in out_copies(g - 1, 1 - p):
                    c.wait()

            @pl.when(g + 1 < NCH)
            def _():
                for c in in_copies(g + 1, 1 - p):
                    c.start()

            @plsc.parallel_loop(0, NVEC, 1, unroll=4)
            def combine(j):
                sl = pl.ds(j * L, L)
                gs = gsc_v[g, sl]
                gc = gsc_v[NCH + g, sl]
                bs = bs_v[0, sl]
                bc = bc_v[0, sl]
                bgs_v[sl] = bs * gc + bc * gs
                bgc_v[sl] = bc * gc - bs * gs

            for c in in_copies(g, p):
                c.wait()

            @plsc.parallel_loop(0, C, 1, unroll=2)
            def row_body(i):
                for j in range(NVEC):
                    sl = pl.ds(j * L, L)
                    e = bgs_v[sl] * rsc_v[C + i, sl] + bgc_v[sl] * rsc_v[i, sl]
                    for b in range(B):
                        plsc.addupdate(rows_v.at[p, b, i, sl], e)

            for c in out_copies(g, p):
                c.start()

        def outer(t, carry):
            for p in range(2):
                step(t * 2 + p, p)
            return carry

        lax.fori_loop(0, NCH // 2, outer, 0)

        for c in out_copies(NCH - 1, (NCH - 1) % 2):
            c.wait()

    return k


def kernel(x, tok_table):
    B, S = x.shape
    V, D = tok_table.shape
    NW = 32
    C = 8
    tab = _pos_factor_table(S, D, NW, C)
    out = _build_sc_kernel(B, S, V, D)(x.astype(jnp.int32), tab, tok_table)
    return out.reshape(B, S, D)


# speedup vs baseline: 1.1147x; 1.1147x over previous
import functools

import numpy as np
import jax
import jax.numpy as jnp
from jax import lax
from jax.experimental import pallas as pl
from jax.experimental.pallas import tpu as pltpu
from jax.experimental.pallas import tpu_sc as plsc


@functools.lru_cache(maxsize=None)
def _pos_factor_table(S: int, D: int, NW: int, C: int):
    SW = S // NW
    NCH = SW // C
    i = np.arange(0, D, 2, dtype=np.float64)
    freq = np.repeat(1.0 / np.power(10000.0, i / D), 2)
    off = np.tile(np.array([0.0, np.pi / 2]), D // 2)
    w = np.arange(NW, dtype=np.float64)[:, None]
    g = np.arange(NCH, dtype=np.float64)[:, None]
    r = np.arange(C, dtype=np.float64)[:, None]
    alpha = w * SW * freq[None, :] + off[None, :]
    beta = g * C * freq[None, :]
    gamma = r * freq[None, :]
    tab = np.concatenate([
        np.sin(alpha), np.cos(alpha),
        np.sin(beta), np.cos(beta),
        np.sin(gamma), np.cos(gamma),
    ]).astype(np.float32)
    return jnp.asarray(tab)


@functools.lru_cache(maxsize=None)
def _build_sc_kernel(B: int, S: int, V: int, D: int):
    info = plsc.get_sparse_core_info()
    NC, NS, L = info.num_cores, info.num_subcores, info.num_lanes
    NW = NC * NS
    SW = S // NW
    C = 8
    NCH = SW // C
    NVEC = D // L

    mesh = plsc.VectorSubcoreMesh(core_axis_name="c", subcore_axis_name="s")

    @functools.partial(
        pl.kernel,
        out_type=jax.ShapeDtypeStruct((B * S, D), jnp.float32),
        mesh=mesh,
        scratch_types=[
            pltpu.VMEM((B, SW), jnp.int32),
            pltpu.VMEM((2, B, C, D), jnp.float32),
            pltpu.VMEM((1, D), jnp.float32),
            pltpu.VMEM((1, D), jnp.float32),
            pltpu.VMEM((2 * NCH, D), jnp.float32),
            pltpu.VMEM((2 * C, D), jnp.float32),
            pltpu.VMEM((D,), jnp.float32),
            pltpu.VMEM((D,), jnp.float32),
            pltpu.VMEM((C, D), jnp.float32),
            pltpu.SemaphoreType.DMA((2,)),
            pltpu.SemaphoreType.DMA((2,)),
        ],
    )
    def k(x_hbm, tab_hbm, table_hbm, out_hbm, idx_v, rows_v,
          bs_v, bc_v, gsc_v, rsc_v, bgs_v, bgc_v, enc_v, gsem, ssem):
        wid = lax.axis_index("s") * NC + lax.axis_index("c")
        s_base = wid * SW
        for b in range(B):
            pltpu.sync_copy(x_hbm.at[b, pl.ds(s_base, SW)], idx_v.at[b])
        pltpu.sync_copy(tab_hbm.at[pl.ds(wid, 1)], bs_v)
        pltpu.sync_copy(tab_hbm.at[pl.ds(NW + wid, 1)], bc_v)
        pltpu.sync_copy(tab_hbm.at[pl.ds(2 * NW, 2 * NCH)], gsc_v)
        pltpu.sync_copy(tab_hbm.at[pl.ds(2 * NW + 2 * NCH, 2 * C)], rsc_v)

        def in_copies(g, p):
            for b in range(B):
                yield pltpu.make_async_copy(
                    table_hbm.at[idx_v.at[b, pl.ds(g * C, C)]],
                    rows_v.at[p, b], gsem.at[p])

        def out_copies(g, p):
            s0 = s_base + g * C
            for b in range(B):
                yield pltpu.make_async_copy(
                    rows_v.at[p, b], out_hbm.at[pl.ds(b * S + s0, C)],
                    ssem.at[p])

        for c in in_copies(0, 0):
            c.start()

        def step(g, p):
            @pl.when(g >= 1)
            def _():
                for c in out_copies(g - 1, 1 - p):
                    c.wait()

            @pl.when(g + 1 < NCH)
            def _():
                for c in in_copies(g + 1, 1 - p):
                    c.start()

            @plsc.parallel_loop(0, NVEC, 1, unroll=4)
            def combine(j):
                sl = pl.ds(j * L, L)
                gs = gsc_v[g, sl]
                gc = gsc_v[NCH + g, sl]
                bs = bs_v[0, sl]
                bc = bc_v[0, sl]
                bgs_v[sl] = bs * gc + bc * gs
                bgc_v[sl] = bc * gc - bs * gs

            @plsc.parallel_loop(0, C, 1, unroll=2)
            def enc_body(i):
                for j in range(NVEC):
                    sl = pl.ds(j * L, L)
                    enc_v[i, sl] = (bgs_v[sl] * rsc_v[C + i, sl]
                                    + bgc_v[sl] * rsc_v[i, sl])

            for c in in_copies(g, p):
                c.wait()

            @plsc.parallel_loop(0, C, 1, unroll=2)
            def row_body(i):
                for j in range(NVEC):
                    sl = pl.ds(j * L, L)
                    e = enc_v[i, sl]
                    for b in range(B):
                        plsc.addupdate(rows_v.at[p, b, i, sl], e)

            for c in out_copies(g, p):
                c.start()

        def outer(t, carry):
            for p in range(2):
                step(t * 2 + p, p)
            return carry

        lax.fori_loop(0, NCH // 2, outer, 0)

        for c in out_copies(NCH - 1, (NCH - 1) % 2):
            c.wait()

    return k


def kernel(x, tok_table):
    B, S = x.shape
    V, D = tok_table.shape
    NW = 32
    C = 8
    tab = _pos_factor_table(S, D, NW, C)
    out = _build_sc_kernel(B, S, V, D)(x.astype(jnp.int32), tab, tok_table)
    return out.reshape(B, S, D)


# speedup vs baseline: 1.6294x; 1.4617x over previous
import functools

import numpy as np
import jax
import jax.numpy as jnp
from jax import lax
from jax.experimental import pallas as pl
from jax.experimental.pallas import tpu as pltpu
from jax.experimental.pallas import tpu_sc as plsc


@functools.lru_cache(maxsize=None)
def _pos_encoding_factors(seq_len: int, d_model: int):
    assert seq_len % 64 == 0
    na = seq_len // 64
    i = np.arange(0, d_model, 2, dtype=np.float64)
    freq = np.repeat(1.0 / np.power(10000.0, i / d_model), 2)
    off = np.tile(np.array([0.0, np.pi / 2]), d_model // 2)
    a = np.arange(na, dtype=np.float64)[:, None]
    b = np.arange(64, dtype=np.float64)[:, None]
    alpha = a * 64.0 * freq[None, :] + off[None, :]
    beta = b * freq[None, :]
    f32 = lambda v: jnp.asarray(v.astype(np.float32))
    return f32(np.sin(alpha)), f32(np.cos(alpha)), f32(np.sin(beta)), f32(np.cos(beta))


def _pos_encoding(seq_len: int, d_model: int):
    sa, ca, sb, cb = _pos_encoding_factors(seq_len, d_model)
    enc = sa[:, None, :] * cb[None, :, :] + ca[:, None, :] * sb[None, :, :]
    return enc.reshape(seq_len, d_model)


@functools.lru_cache(maxsize=None)
def _build_sc_kernel(B: int, S: int, V: int, D: int):
    info = plsc.get_sparse_core_info()
    NC, NS, L = info.num_cores, info.num_subcores, info.num_lanes
    NW = NC * NS
    SW = S // NW
    C = 8
    NCH = SW // C
    G = B * C
    NVEC = D // L

    mesh = plsc.VectorSubcoreMesh(core_axis_name="c", subcore_axis_name="s")

    @functools.partial(
        pl.kernel,
        out_type=jax.ShapeDtypeStruct((B * S, D), jnp.float32),
        mesh=mesh,
        scratch_types=[
            pltpu.VMEM((1, NCH * G), jnp.int32),
            pltpu.VMEM((2, C, D), jnp.float32),
            pltpu.VMEM((2, G, D), jnp.float32),
            pltpu.SemaphoreType.DMA((2,)),
            pltpu.SemaphoreType.DMA((2,)),
        ],
    )
    def k(xt_hbm, enc_hbm, table_hbm, out_hbm, idx_v, enc_v, rows_v,
          gsem, ssem):
        wid = lax.axis_index("s") * NC + lax.axis_index("c")
        s_base = wid * SW
        pltpu.sync_copy(xt_hbm.at[pl.ds(wid, 1)], idx_v)

        def in_copies(g, p):
            s0 = s_base + g * C
            yield pltpu.make_async_copy(
                enc_hbm.at[pl.ds(s0, C)], enc_v.at[p], gsem.at[p])
            yield pltpu.make_async_copy(
                table_hbm.at[idx_v.at[0, pl.ds(g * G, G)]],
                rows_v.at[p], gsem.at[p])

        def out_copies(g, p):
            s0 = s_base + g * C
            for b in range(B):
                yield pltpu.make_async_copy(
                    rows_v.at[p, pl.ds(b * C, C)],
                    out_hbm.at[pl.ds(b * S + s0, C)], ssem.at[p])

        for c in in_copies(0, 0):
            c.start()

        def step(g, p):
            @pl.when(g >= 1)
            def _():
                for c in out_copies(g - 1, 1 - p):
                    c.wait()

            @pl.when(g + 1 < NCH)
            def _():
                for c in in_copies(g + 1, 1 - p):
                    c.start()

            for c in in_copies(g, p):
                c.wait()

            @plsc.parallel_loop(0, C, 1, unroll=2)
            def row_body(i):
                for j in range(NVEC):
                    sl = pl.ds(j * L, L)
                    e = enc_v[p, i, sl]
                    for b in range(B):
                        plsc.addupdate(rows_v.at[p, b * C + i, sl], e)

            for c in out_copies(g, p):
                c.start()

        def outer(t, carry):
            for p in range(2):
                step(t * 2 + p, p)
            return carry

        lax.fori_loop(0, NCH // 2, outer, 0)

        for c in out_copies(NCH - 1, (NCH - 1) % 2):
            c.wait()

    return k


def kernel(x, tok_table):
    B, S = x.shape
    V, D = tok_table.shape
    NW = 32
    SW = S // NW
    C = 8
    enc = _pos_encoding(S, D)
    xt = (x.astype(jnp.int32)
          .reshape(B, NW, SW // C, C)
          .transpose(1, 2, 0, 3)
          .reshape(NW, SW * B))
    out = _build_sc_kernel(B, S, V, D)(xt, enc, tok_table)
    return out.reshape(B, S, D)


# speedup vs baseline: 1.6638x; 1.0211x over previous
import functools

import numpy as np
import jax
import jax.numpy as jnp
from jax import lax
from jax.experimental import pallas as pl
from jax.experimental.pallas import tpu as pltpu
from jax.experimental.pallas import tpu_sc as plsc


@functools.lru_cache(maxsize=None)
def _pos_encoding_factors(seq_len: int, d_model: int):
    assert seq_len % 64 == 0
    na = seq_len // 64
    i = np.arange(0, d_model, 2, dtype=np.float64)
    freq = np.repeat(1.0 / np.power(10000.0, i / d_model), 2)
    off = np.tile(np.array([0.0, np.pi / 2]), d_model // 2)
    a = np.arange(na, dtype=np.float64)[:, None]
    b = np.arange(64, dtype=np.float64)[:, None]
    alpha = a * 64.0 * freq[None, :] + off[None, :]
    beta = b * freq[None, :]
    f32 = lambda v: jnp.asarray(v.astype(np.float32))
    return f32(np.sin(alpha)), f32(np.cos(alpha)), f32(np.sin(beta)), f32(np.cos(beta))


def _pos_encoding(seq_len: int, d_model: int):
    sa, ca, sb, cb = _pos_encoding_factors(seq_len, d_model)
    enc = sa[:, None, :] * cb[None, :, :] + ca[:, None, :] * sb[None, :, :]
    return enc.reshape(seq_len, d_model)


@functools.lru_cache(maxsize=None)
def _build_sc_kernel(B: int, S: int, V: int, D: int):
    info = plsc.get_sparse_core_info()
    NC, NS, L = info.num_cores, info.num_subcores, info.num_lanes
    NW = NC * NS
    SW = S // NW
    C = 8
    NCH = SW // C
    NVEC = D // L

    mesh = plsc.VectorSubcoreMesh(core_axis_name="c", subcore_axis_name="s")

    @functools.partial(
        pl.kernel,
        out_type=jax.ShapeDtypeStruct((B * S, D), jnp.float32),
        mesh=mesh,
        scratch_types=[
            pltpu.VMEM((B, SW), jnp.int32),
            pltpu.VMEM((2, C, D), jnp.float32),
            pltpu.VMEM((2, B, C, D), jnp.float32),
            pltpu.SemaphoreType.DMA((2,)),
            pltpu.SemaphoreType.DMA((2,)),
        ],
    )
    def k(x_hbm, enc_hbm, table_hbm, out_hbm, idx_v, enc_v, rows_v, gsem, ssem):
        wid = lax.axis_index("s") * NC + lax.axis_index("c")
        s_base = wid * SW
        for b in range(B):
            pltpu.sync_copy(x_hbm.at[b, pl.ds(s_base, SW)], idx_v.at[b])

        def in_copies(g, p):
            s0 = s_base + g * C
            yield pltpu.make_async_copy(
                enc_hbm.at[pl.ds(s0, C)], enc_v.at[p], gsem.at[p])
            for b in range(B):
                yield pltpu.make_async_copy(
                    table_hbm.at[idx_v.at[b, pl.ds(g * C, C)]],
                    rows_v.at[p, b], gsem.at[p])

        def out_copies(g, p):
            s0 = s_base + g * C
            for b in range(B):
                yield pltpu.make_async_copy(
                    rows_v.at[p, b], out_hbm.at[pl.ds(b * S + s0, C)],
                    ssem.at[p])

        for c in in_copies(0, 0):
            c.start()

        def step(g, p):
            @pl.when(g >= 1)
            def _():
                for c in out_copies(g - 1, 1 - p):
                    c.wait()

            @pl.when(g + 1 < NCH)
            def _():
                for c in in_copies(g + 1, 1 - p):
                    c.start()

            for c in in_copies(g, p):
                c.wait()

            @plsc.parallel_loop(0, C, 1, unroll=2)
            def row_body(i):
                for j in range(NVEC):
                    sl = pl.ds(j * L, L)
                    e = enc_v[p, i, sl]
                    for b in range(B):
                        plsc.addupdate(rows_v.at[p, b, i, sl], e)

            for c in out_copies(g, p):
                c.start()

        def outer(t, carry):
            for p in range(2):
                step(t * 2 + p, p)
            return carry

        lax.fori_loop(0, NCH // 2, outer, 0)

        for c in out_copies(NCH - 1, (NCH - 1) % 2):
            c.wait()

    return k


def kernel(x, tok_table):
    B, S = x.shape
    V, D = tok_table.shape
    enc = _pos_encoding(S, D)
    out = _build_sc_kernel(B, S, V, D)(x.astype(jnp.int32), enc, tok_table)
    return out.reshape(B, S, D)
